# baseline (speedup 1.0000x reference)
import functools

import jax
import jax.numpy as jnp
from jax import lax
from jax.experimental import pallas as pl
from jax.experimental.pallas import tpu as pltpu
from jax.experimental.pallas import tpu_sc as plsc

B, N, K, H = 2, 2048, 32, 128
FF = 4 * H
SCALE = 30.0
M = B * N * K

_NW = 32
_RPW = M // _NW
_CH = 128
_NCHUNK = _RPW // _CH


def _sc_gather(table, idx):
    info = plsc.get_sparse_core_info()
    nc = info.num_cores
    mesh = plsc.VectorSubcoreMesh(core_axis_name="c", subcore_axis_name="s")

    @functools.partial(
        pl.kernel,
        mesh=mesh,
        out_type=jax.ShapeDtypeStruct((M, H), jnp.float32),
        scratch_types=[
            pltpu.VMEM((_RPW,), jnp.int32),
            pltpu.VMEM((_CH, H), jnp.float32),
            pltpu.VMEM((_CH, H), jnp.float32),
            pltpu.SemaphoreType.DMA,
            pltpu.SemaphoreType.DMA,
            pltpu.SemaphoreType.DMA,
            pltpu.SemaphoreType.DMA,
        ],
    )
    def gk(table_hbm, idx_hbm, out_hbm, idx_all, rows0, rows1,
           sg0, sg1, so0, so1):
        wid = lax.axis_index("s") * nc + lax.axis_index("c")
        base = wid * _RPW
        boff = (base // (N * K)) * N

        pltpu.sync_copy(idx_hbm.at[pl.ds(base, _RPW)], idx_all)

        def addb(i, c2):
            sl = pl.ds(i * 16, 16)
            idx_all[sl] = idx_all[sl] + boff
            return c2

        lax.fori_loop(0, _RPW // 16, addb, 0)

        def pair(i, first):
            e_off = (2 * i) * _CH
            o_off = e_off + _CH
            if not first:
                pltpu.make_async_copy(rows0, out_hbm.at[pl.ds(0, _CH)], so0).wait()
            pltpu.async_copy(
                table_hbm.at[idx_all.at[pl.ds(e_off, _CH)]], rows0, sg0)
            if not first:
                pltpu.make_async_copy(rows1, out_hbm.at[pl.ds(0, _CH)], so1).wait()
            pltpu.async_copy(
                table_hbm.at[idx_all.at[pl.ds(o_off, _CH)]], rows1, sg1)
            pltpu.make_async_copy(
                table_hbm.at[idx_all.at[pl.ds(e_off, _CH)]], rows0, sg0).wait()
            pltpu.async_copy(rows0, out_hbm.at[pl.ds(base + e_off, _CH)], so0)
            pltpu.make_async_copy(
                table_hbm.at[idx_all.at[pl.ds(o_off, _CH)]], rows1, sg1).wait()
            pltpu.async_copy(rows1, out_hbm.at[pl.ds(base + o_off, _CH)], so1)

        pair(0, True)

        def body(i, carry):
            pair(i, False)
            return carry

        lax.fori_loop(1, _NCHUNK // 2, body, 0)
        pltpu.make_async_copy(rows0, out_hbm.at[pl.ds(0, _CH)], so0).wait()
        pltpu.make_async_copy(rows1, out_hbm.at[pl.ds(0, _CH)], so1).wait()

    return gk(table, idx)


def _gelu(x):
    return x * 0.5 * (1.0 + lax.erf(x * 0.7071067811865476))


def _ln(x, g, b):
    m = jnp.mean(x, axis=-1, keepdims=True)
    v = jnp.mean((x - m) ** 2, axis=-1, keepdims=True)
    return (x - m) * lax.rsqrt(v + 1e-5) * g + b


def _ln_mxu(x, g, b):
    o = jnp.full((H, H), 1.0 / H, jnp.float32)
    m = jnp.dot(x, o, preferred_element_type=jnp.float32)
    d = x - m
    v = jnp.dot(d * d, o, preferred_element_type=jnp.float32)
    return d * lax.rsqrt(v + 1e-5) * g + b


_BN = 256
_BE = _BN * K


def _proj_body(x_ref, w_ref, out_ref):
    out_ref[...] = jnp.dot(x_ref[...], w_ref[...],
                           preferred_element_type=jnp.float32)


def _tc1_body(hv_ref, he_ref, g_ref, w1a_ref, w1b_ref, b1_ref,
              w2_ref, b2_ref, w3_ref, b3_ref, win_ref, bin_ref, wout_ref,
              bout_ref, g1_ref, gb1_ref, g2_ref, gb2_ref, w11c_ref,
              out_ref, p2_ref):
    v = hv_ref[...]
    a = jnp.dot(v, w1a_ref[...], preferred_element_type=jnp.float32) + b1_ref[...]
    t = jnp.dot(he_ref[...], w1b_ref[...], preferred_element_type=jnp.float32) + g_ref[...]
    t = t.reshape(_BN, K, H) + a[:, None, :]
    t1 = _gelu(t).reshape(_BE, H)
    t2 = _gelu(jnp.dot(t1, w2_ref[...], preferred_element_type=jnp.float32) + b2_ref[...])
    msg = jnp.dot(t2, w3_ref[...], preferred_element_type=jnp.float32) + b3_ref[...]
    dh = jnp.sum(msg.reshape(_BN, K, H), axis=1) * (1.0 / SCALE)
    x = _ln(v + dh, g1_ref[...], gb1_ref[...])
    f = _gelu(jnp.dot(x, win_ref[...], preferred_element_type=jnp.float32) + bin_ref[...])
    y = x + jnp.dot(f, wout_ref[...], preferred_element_type=jnp.float32) + bout_ref[...]
    y = _ln(y, g2_ref[...], gb2_ref[...])
    out_ref[...] = y
    p2_ref[...] = jnp.dot(y, w11c_ref[...], preferred_element_type=jnp.float32)


def _tc2_body(hv_ref, he_ref, g_ref, w1a_ref, w1b_ref, b1_ref,
              w2_ref, b2_ref, w3_ref, b3_ref, g3_ref, gb3_ref, out_ref):
    v = hv_ref[...]
    e = he_ref[...]
    a = jnp.dot(v, w1a_ref[...], preferred_element_type=jnp.float32) + b1_ref[...]
    t = jnp.dot(e, w1b_ref[...], preferred_element_type=jnp.float32) + g_ref[...]
    t = t.reshape(_BN, K, H) + a[:, None, :]
    t1 = _gelu(t).reshape(_BE, H)
    t2 = _gelu(jnp.dot(t1, w2_ref[...], preferred_element_type=jnp.float32) + b2_ref[...])
    msg = jnp.dot(t2, w3_ref[...], preferred_element_type=jnp.float32) + b3_ref[...]
    out_ref[...] = _ln_mxu(e + msg, g3_ref[...], gb3_ref[...])


def _node_spec():
    return pl.BlockSpec((_BN, H), lambda i: (i, 0))


def _edge_spec():
    return pl.BlockSpec((_BE, H), lambda i: (i, 0))


def _w_spec(shape):
    return pl.BlockSpec(shape, lambda i: (0,) * len(shape))


def kernel(h_V, h_E, E_idx, W1_w, W1_b, W2_w, W2_b, W3_w, W3_b, W11_w, W11_b,
           W12_w, W12_b, W13_w, W13_b, Win_w, Win_b, Wout_w, Wout_b,
           norm1_g, norm1_b, norm2_g, norm2_b, norm3_g, norm3_b):
    hv2 = h_V.reshape(B * N, H)
    he2 = h_E.reshape(M, H)
    idx = E_idx.reshape(M)

    w1a, w1b, w1c = W1_w[:H], W1_w[H:2 * H], W1_w[2 * H:]
    w11a, w11b, w11c = W11_w[:H], W11_w[H:2 * H], W11_w[2 * H:]

    row = lambda x: x.reshape(1, -1)
    grid = (B * N) // _BN

    p1 = pl.pallas_call(
        _proj_body,
        grid=(4,),
        in_specs=[pl.BlockSpec((B * N // 4, H), lambda i: (i, 0)), _w_spec((H, H))],
        out_specs=pl.BlockSpec((B * N // 4, H), lambda i: (i, 0)),
        out_shape=jax.ShapeDtypeStruct((B * N, H), jnp.float32),
    )(hv2, w1c)

    g1 = _sc_gather(p1, idx)

    wspecs1 = [
        _w_spec((H, H)), _w_spec((H, H)), _w_spec((1, H)),
        _w_spec((H, H)), _w_spec((1, H)), _w_spec((H, H)), _w_spec((1, H)),
        _w_spec((H, FF)), _w_spec((1, FF)), _w_spec((FF, H)), _w_spec((1, H)),
        _w_spec((1, H)), _w_spec((1, H)), _w_spec((1, H)), _w_spec((1, H)),
        _w_spec((H, H)),
    ]
    hv_new, p2 = pl.pallas_call(
        _tc1_body,
        grid=(grid,),
        in_specs=[_node_spec(), _edge_spec(), _edge_spec()] + wspecs1,
        out_specs=[_node_spec(), _node_spec()],
        out_shape=[jax.ShapeDtypeStruct((B * N, H), jnp.float32),
                   jax.ShapeDtypeStruct((B * N, H), jnp.float32)],
    )(hv2, he2, g1, w1a, w1b, row(W1_b), W2_w, row(W2_b), W3_w,
      row(W3_b), Win_w, row(Win_b), Wout_w, row(Wout_b), row(norm1_g),
      row(norm1_b), row(norm2_g), row(norm2_b), w11c)

    g2 = _sc_gather(p2, idx)

    wspecs2 = [
        _w_spec((H, H)), _w_spec((H, H)), _w_spec((1, H)),
        _w_spec((H, H)), _w_spec((1, H)), _w_spec((H, H)), _w_spec((1, H)),
        _w_spec((1, H)), _w_spec((1, H)),
    ]
    he_new = pl.pallas_call(
        _tc2_body,
        grid=(grid,),
        in_specs=[_node_spec(), _edge_spec(), _edge_spec()] + wspecs2,
        out_specs=_edge_spec(),
        out_shape=jax.ShapeDtypeStruct((M, H), jnp.float32),
    )(hv_new, he2, g2, w11a, w11b, row(W11_b), W12_w, row(W12_b),
      W13_w, row(W13_b), row(norm3_g), row(norm3_b))

    return hv_new.reshape(B, N, H), he_new.reshape(B, N, K, H)


# speedup vs baseline: 1.0171x; 1.0171x over previous
import functools

import jax
import jax.numpy as jnp
from jax import lax
from jax.experimental import pallas as pl
from jax.experimental.pallas import tpu as pltpu
from jax.experimental.pallas import tpu_sc as plsc

B, N, K, H = 2, 2048, 32, 128
FF = 4 * H
SCALE = 30.0
M = B * N * K

_NW = 32
_RPW = M // _NW
_CH = 128
_NCHUNK = _RPW // _CH


def _sc_gather(table, idx):
    info = plsc.get_sparse_core_info()
    nc = info.num_cores
    mesh = plsc.VectorSubcoreMesh(core_axis_name="c", subcore_axis_name="s")

    @functools.partial(
        pl.kernel,
        mesh=mesh,
        out_type=jax.ShapeDtypeStruct((M, H), jnp.float32),
        scratch_types=[
            pltpu.VMEM((_CH,), jnp.int32),
            pltpu.VMEM((_CH,), jnp.int32),
            pltpu.VMEM((_CH, H), jnp.float32),
            pltpu.VMEM((_CH, H), jnp.float32),
            pltpu.SemaphoreType.DMA,
            pltpu.SemaphoreType.DMA,
            pltpu.SemaphoreType.DMA,
            pltpu.SemaphoreType.DMA,
        ],
    )
    def gk(table_hbm, idx_hbm, out_hbm, idx0, idx1, rows0, rows1,
           sg0, sg1, so0, so1):
        wid = lax.axis_index("s") * nc + lax.axis_index("c")
        base = wid * _RPW
        boff = (base // (N * K)) * N

        def load_idx(off, idx_v):
            pltpu.sync_copy(idx_hbm.at[pl.ds(off, _CH)], idx_v)

            def addb(i, c2):
                sl = pl.ds(i * 16, 16)
                idx_v[sl] = idx_v[sl] + boff
                return c2

            lax.fori_loop(0, _CH // 16, addb, 0, unroll=True)

        def pair(i, first):
            e_off = base + (2 * i) * _CH
            o_off = e_off + _CH
            load_idx(e_off, idx0)
            if not first:
                pltpu.make_async_copy(rows0, out_hbm.at[pl.ds(0, _CH)], so0).wait()
            pltpu.async_copy(table_hbm.at[idx0], rows0, sg0)
            load_idx(o_off, idx1)
            if not first:
                pltpu.make_async_copy(rows1, out_hbm.at[pl.ds(0, _CH)], so1).wait()
            pltpu.async_copy(table_hbm.at[idx1], rows1, sg1)
            pltpu.make_async_copy(table_hbm.at[idx0], rows0, sg0).wait()
            pltpu.async_copy(rows0, out_hbm.at[pl.ds(e_off, _CH)], so0)
            pltpu.make_async_copy(table_hbm.at[idx1], rows1, sg1).wait()
            pltpu.async_copy(rows1, out_hbm.at[pl.ds(o_off, _CH)], so1)

        pair(0, True)

        def body(i, carry):
            pair(i, False)
            return carry

        lax.fori_loop(1, _NCHUNK // 2, body, 0)
        pltpu.make_async_copy(rows0, out_hbm.at[pl.ds(0, _CH)], so0).wait()
        pltpu.make_async_copy(rows1, out_hbm.at[pl.ds(0, _CH)], so1).wait()

    return gk(table, idx)


def _gelu(x):
    return x * 0.5 * (1.0 + lax.erf(x * 0.7071067811865476))


def _ln(x, g, b):
    m = jnp.mean(x, axis=-1, keepdims=True)
    v = jnp.mean((x - m) ** 2, axis=-1, keepdims=True)
    return (x - m) * lax.rsqrt(v + 1e-5) * g + b


def _ln_mxu(x, g, b):
    o = jnp.full((H, H), 1.0 / H, jnp.float32)
    m = jnp.dot(x, o, preferred_element_type=jnp.float32)
    d = x - m
    v = jnp.dot(d * d, o, preferred_element_type=jnp.float32)
    return d * lax.rsqrt(v + 1e-5) * g + b


_BN = 256
_BE = _BN * K
_BN1 = 512
_BE1 = _BN1 * K


def _proj_body(x_ref, w_ref, out_ref):
    out_ref[...] = jnp.dot(x_ref[...], w_ref[...],
                           preferred_element_type=jnp.float32)


def _tc1_body(hv_ref, he_ref, g_ref, w1a_ref, w1b_ref, b1_ref,
              w2_ref, b2_ref, w3_ref, b3_ref, win_ref, bin_ref, wout_ref,
              bout_ref, g1_ref, gb1_ref, g2_ref, gb2_ref, w11c_ref,
              out_ref, p2_ref):
    v = hv_ref[...]
    a = jnp.dot(v, w1a_ref[...], preferred_element_type=jnp.float32) + b1_ref[...]
    t = jnp.dot(he_ref[...], w1b_ref[...], preferred_element_type=jnp.float32) + g_ref[...]
    t = t.reshape(_BN1, K, H) + a[:, None, :]
    t1 = _gelu(t).reshape(_BE1, H)
    t2 = _gelu(jnp.dot(t1, w2_ref[...], preferred_element_type=jnp.float32) + b2_ref[...])
    msg = jnp.dot(t2, w3_ref[...], preferred_element_type=jnp.float32) + b3_ref[...]
    dh = jnp.sum(msg.reshape(_BN1, K, H), axis=1) * (1.0 / SCALE)
    x = _ln(v + dh, g1_ref[...], gb1_ref[...])
    f = _gelu(jnp.dot(x, win_ref[...], preferred_element_type=jnp.float32) + bin_ref[...])
    y = x + jnp.dot(f, wout_ref[...], preferred_element_type=jnp.float32) + bout_ref[...]
    y = _ln(y, g2_ref[...], gb2_ref[...])
    out_ref[...] = y
    p2_ref[...] = jnp.dot(y, w11c_ref[...], preferred_element_type=jnp.float32)


def _tc2_body(hv_ref, he_ref, g_ref, w1a_ref, w1b_ref, b1_ref,
              w2_ref, b2_ref, w3_ref, b3_ref, g3_ref, gb3_ref, out_ref):
    v = hv_ref[...]
    e = he_ref[...]
    a = jnp.dot(v, w1a_ref[...], preferred_element_type=jnp.float32) + b1_ref[...]
    t = jnp.dot(e, w1b_ref[...], preferred_element_type=jnp.float32) + g_ref[...]
    t = t.reshape(_BN, K, H) + a[:, None, :]
    t1 = _gelu(t).reshape(_BE, H)
    t2 = _gelu(jnp.dot(t1, w2_ref[...], preferred_element_type=jnp.float32) + b2_ref[...])
    msg = jnp.dot(t2, w3_ref[...], preferred_element_type=jnp.float32) + b3_ref[...]
    out_ref[...] = _ln_mxu(e + msg, g3_ref[...], gb3_ref[...])


def _node_spec():
    return pl.BlockSpec((_BN, H), lambda i: (i, 0))


def _node1_spec():
    return pl.BlockSpec((_BN1, H), lambda i: (i, 0))


def _edge1_spec():
    return pl.BlockSpec((_BE1, H), lambda i: (i, 0))


def _edge_spec():
    return pl.BlockSpec((_BE, H), lambda i: (i, 0))


def _w_spec(shape):
    return pl.BlockSpec(shape, lambda i: (0,) * len(shape))


def kernel(h_V, h_E, E_idx, W1_w, W1_b, W2_w, W2_b, W3_w, W3_b, W11_w, W11_b,
           W12_w, W12_b, W13_w, W13_b, Win_w, Win_b, Wout_w, Wout_b,
           norm1_g, norm1_b, norm2_g, norm2_b, norm3_g, norm3_b):
    hv2 = h_V.reshape(B * N, H)
    he2 = h_E.reshape(M, H)
    idx = E_idx.reshape(M)

    w1a, w1b, w1c = W1_w[:H], W1_w[H:2 * H], W1_w[2 * H:]
    w11a, w11b, w11c = W11_w[:H], W11_w[H:2 * H], W11_w[2 * H:]

    row = lambda x: x.reshape(1, -1)
    grid = (B * N) // _BN

    p1 = pl.pallas_call(
        _proj_body,
        grid=(4,),
        in_specs=[pl.BlockSpec((B * N // 4, H), lambda i: (i, 0)), _w_spec((H, H))],
        out_specs=pl.BlockSpec((B * N // 4, H), lambda i: (i, 0)),
        out_shape=jax.ShapeDtypeStruct((B * N, H), jnp.float32),
    )(hv2, w1c)

    g1 = _sc_gather(p1, idx)

    wspecs1 = [
        _w_spec((H, H)), _w_spec((H, H)), _w_spec((1, H)),
        _w_spec((H, H)), _w_spec((1, H)), _w_spec((H, H)), _w_spec((1, H)),
        _w_spec((H, FF)), _w_spec((1, FF)), _w_spec((FF, H)), _w_spec((1, H)),
        _w_spec((1, H)), _w_spec((1, H)), _w_spec((1, H)), _w_spec((1, H)),
        _w_spec((H, H)),
    ]
    hv_new, p2 = pl.pallas_call(
        _tc1_body,
        grid=((B * N) // _BN1,),
        in_specs=[_node1_spec(), _edge1_spec(), _edge1_spec()] + wspecs1,
        out_specs=[_node1_spec(), _node1_spec()],
        out_shape=[jax.ShapeDtypeStruct((B * N, H), jnp.float32),
                   jax.ShapeDtypeStruct((B * N, H), jnp.float32)],
    )(hv2, he2, g1, w1a, w1b, row(W1_b), W2_w, row(W2_b), W3_w,
      row(W3_b), Win_w, row(Win_b), Wout_w, row(Wout_b), row(norm1_g),
      row(norm1_b), row(norm2_g), row(norm2_b), w11c)

    g2 = _sc_gather(p2, idx)

    wspecs2 = [
        _w_spec((H, H)), _w_spec((H, H)), _w_spec((1, H)),
        _w_spec((H, H)), _w_spec((1, H)), _w_spec((H, H)), _w_spec((1, H)),
        _w_spec((1, H)), _w_spec((1, H)),
    ]
    he_new = pl.pallas_call(
        _tc2_body,
        grid=(grid,),
        in_specs=[_node_spec(), _edge_spec(), _edge_spec()] + wspecs2,
        out_specs=_edge_spec(),
        out_shape=jax.ShapeDtypeStruct((M, H), jnp.float32),
    )(hv_new, he2, g2, w11a, w11b, row(W11_b), W12_w, row(W12_b),
      W13_w, row(W13_b), row(norm3_g), row(norm3_b))

    return hv_new.reshape(B, N, H), he_new.reshape(B, N, K, H)


# speedup vs baseline: 1.0477x; 1.0300x over previous
import functools

import jax
import jax.numpy as jnp
from jax import lax
from jax.experimental import pallas as pl
from jax.experimental.pallas import tpu as pltpu
from jax.experimental.pallas import tpu_sc as plsc

B, N, K, H = 2, 2048, 32, 128
FF = 4 * H
SCALE = 30.0
M = B * N * K

_NW = 32
_RPW = M // _NW
_CH = 128
_NCHUNK = _RPW // _CH


def _sc_gather(table, idx):
    info = plsc.get_sparse_core_info()
    nc = info.num_cores
    mesh = plsc.VectorSubcoreMesh(core_axis_name="c", subcore_axis_name="s")

    @functools.partial(
        pl.kernel,
        mesh=mesh,
        out_type=jax.ShapeDtypeStruct((M, H), jnp.float32),
        scratch_types=[
            pltpu.VMEM((_CH,), jnp.int32),
            pltpu.VMEM((_CH,), jnp.int32),
            pltpu.VMEM((_CH, H), jnp.float32),
            pltpu.VMEM((_CH, H), jnp.float32),
            pltpu.SemaphoreType.DMA,
            pltpu.SemaphoreType.DMA,
            pltpu.SemaphoreType.DMA,
            pltpu.SemaphoreType.DMA,
        ],
    )
    def gk(table_hbm, idx_hbm, out_hbm, idx0, idx1, rows0, rows1,
           sg0, sg1, so0, so1):
        wid = lax.axis_index("s") * nc + lax.axis_index("c")
        base = wid * _RPW
        boff = (base // (N * K)) * N

        def load_idx(off, idx_v):
            pltpu.sync_copy(idx_hbm.at[pl.ds(off, _CH)], idx_v)

            def addb(i, c2):
                sl = pl.ds(i * 16, 16)
                idx_v[sl] = idx_v[sl] + boff
                return c2

            lax.fori_loop(0, _CH // 16, addb, 0, unroll=True)

        def pair(i, first):
            e_off = base + (2 * i) * _CH
            o_off = e_off + _CH
            load_idx(e_off, idx0)
            if not first:
                pltpu.make_async_copy(rows0, out_hbm.at[pl.ds(0, _CH)], so0).wait()
            pltpu.async_copy(table_hbm.at[idx0], rows0, sg0)
            load_idx(o_off, idx1)
            if not first:
                pltpu.make_async_copy(rows1, out_hbm.at[pl.ds(0, _CH)], so1).wait()
            pltpu.async_copy(table_hbm.at[idx1], rows1, sg1)
            pltpu.make_async_copy(table_hbm.at[idx0], rows0, sg0).wait()
            pltpu.async_copy(rows0, out_hbm.at[pl.ds(e_off, _CH)], so0)
            pltpu.make_async_copy(table_hbm.at[idx1], rows1, sg1).wait()
            pltpu.async_copy(rows1, out_hbm.at[pl.ds(o_off, _CH)], so1)

        pair(0, True)

        def body(i, carry):
            pair(i, False)
            return carry

        lax.fori_loop(1, _NCHUNK // 2, body, 0)
        pltpu.make_async_copy(rows0, out_hbm.at[pl.ds(0, _CH)], so0).wait()
        pltpu.make_async_copy(rows1, out_hbm.at[pl.ds(0, _CH)], so1).wait()

    return gk(table, idx)


def _gelu(x):
    return x * 0.5 * (1.0 + lax.erf(x * 0.7071067811865476))


def _ln(x, g, b):
    m = jnp.mean(x, axis=-1, keepdims=True)
    v = jnp.mean((x - m) ** 2, axis=-1, keepdims=True)
    return (x - m) * lax.rsqrt(v + 1e-5) * g + b


def _ln_mxu(x, g, b):
    o = jnp.full((H, H), 1.0 / H, jnp.float32)
    m = jnp.dot(x, o, preferred_element_type=jnp.float32)
    d = x - m
    v = jnp.dot(d * d, o, preferred_element_type=jnp.float32)
    return d * lax.rsqrt(v + 1e-5) * g + b


_BN = 256
_BE = _BN * K
_BN1 = 512
_BE1 = _BN1 * K


def _proj_body(x_ref, w_ref, out_ref):
    out_ref[...] = jnp.dot(x_ref[...], w_ref[...],
                           preferred_element_type=jnp.float32)


def _tc1_body(hv_ref, he_ref, g_ref, w1a_ref, w1b_ref, b1_ref,
              w2_ref, b2_ref, w3_ref, b3_ref, win_ref, bin_ref, wout_ref,
              bout_ref, g1_ref, gb1_ref, g2_ref, gb2_ref, w11c_ref,
              out_ref, p2_ref):
    v = hv_ref[...]
    a = jnp.dot(v, w1a_ref[...], preferred_element_type=jnp.float32) + b1_ref[...]
    t = jnp.dot(he_ref[...], w1b_ref[...], preferred_element_type=jnp.float32) + g_ref[...]
    t = t.reshape(_BN1, K, H) + a[:, None, :]
    t1 = _gelu(t).reshape(_BE1, H)
    t2 = _gelu(jnp.dot(t1, w2_ref[...], preferred_element_type=jnp.float32) + b2_ref[...])
    s2 = jnp.sum(t2.reshape(_BN1, K, H), axis=1)
    dh = (jnp.dot(s2, w3_ref[...], preferred_element_type=jnp.float32)
          + b3_ref[...] * K) * (1.0 / SCALE)
    x = _ln(v + dh, g1_ref[...], gb1_ref[...])
    f = _gelu(jnp.dot(x, win_ref[...], preferred_element_type=jnp.float32) + bin_ref[...])
    y = x + jnp.dot(f, wout_ref[...], preferred_element_type=jnp.float32) + bout_ref[...]
    y = _ln(y, g2_ref[...], gb2_ref[...])
    out_ref[...] = y
    p2_ref[...] = jnp.dot(y, w11c_ref[...], preferred_element_type=jnp.float32)


def _tc2_body(hv_ref, he_ref, g_ref, w1a_ref, w1b_ref, b1_ref,
              w2_ref, b2_ref, w3_ref, b3_ref, g3_ref, gb3_ref, out_ref):
    v = hv_ref[...]
    e = he_ref[...]
    a = jnp.dot(v, w1a_ref[...], preferred_element_type=jnp.float32) + b1_ref[...]
    t = jnp.dot(e, w1b_ref[...], preferred_element_type=jnp.float32) + g_ref[...]
    t = t.reshape(_BN, K, H) + a[:, None, :]
    t1 = _gelu(t).reshape(_BE, H)
    t2 = _gelu(jnp.dot(t1, w2_ref[...], preferred_element_type=jnp.float32) + b2_ref[...])
    msg = jnp.dot(t2, w3_ref[...], preferred_element_type=jnp.float32) + b3_ref[...]
    out_ref[...] = _ln_mxu(e + msg, g3_ref[...], gb3_ref[...])


def _node_spec():
    return pl.BlockSpec((_BN, H), lambda i: (i, 0))


def _node1_spec():
    return pl.BlockSpec((_BN1, H), lambda i: (i, 0))


def _edge1_spec():
    return pl.BlockSpec((_BE1, H), lambda i: (i, 0))


def _edge_spec():
    return pl.BlockSpec((_BE, H), lambda i: (i, 0))


def _w_spec(shape):
    return pl.BlockSpec(shape, lambda i: (0,) * len(shape))


def kernel(h_V, h_E, E_idx, W1_w, W1_b, W2_w, W2_b, W3_w, W3_b, W11_w, W11_b,
           W12_w, W12_b, W13_w, W13_b, Win_w, Win_b, Wout_w, Wout_b,
           norm1_g, norm1_b, norm2_g, norm2_b, norm3_g, norm3_b):
    hv2 = h_V.reshape(B * N, H)
    he2 = h_E.reshape(M, H)
    idx = E_idx.reshape(M)

    w1a, w1b, w1c = W1_w[:H], W1_w[H:2 * H], W1_w[2 * H:]
    w11a, w11b, w11c = W11_w[:H], W11_w[H:2 * H], W11_w[2 * H:]

    row = lambda x: x.reshape(1, -1)
    grid = (B * N) // _BN

    p1 = pl.pallas_call(
        _proj_body,
        grid=(4,),
        in_specs=[pl.BlockSpec((B * N // 4, H), lambda i: (i, 0)), _w_spec((H, H))],
        out_specs=pl.BlockSpec((B * N // 4, H), lambda i: (i, 0)),
        out_shape=jax.ShapeDtypeStruct((B * N, H), jnp.float32),
    )(hv2, w1c)

    g1 = _sc_gather(p1, idx)

    wspecs1 = [
        _w_spec((H, H)), _w_spec((H, H)), _w_spec((1, H)),
        _w_spec((H, H)), _w_spec((1, H)), _w_spec((H, H)), _w_spec((1, H)),
        _w_spec((H, FF)), _w_spec((1, FF)), _w_spec((FF, H)), _w_spec((1, H)),
        _w_spec((1, H)), _w_spec((1, H)), _w_spec((1, H)), _w_spec((1, H)),
        _w_spec((H, H)),
    ]
    hv_new, p2 = pl.pallas_call(
        _tc1_body,
        grid=((B * N) // _BN1,),
        in_specs=[_node1_spec(), _edge1_spec(), _edge1_spec()] + wspecs1,
        out_specs=[_node1_spec(), _node1_spec()],
        out_shape=[jax.ShapeDtypeStruct((B * N, H), jnp.float32),
                   jax.ShapeDtypeStruct((B * N, H), jnp.float32)],
    )(hv2, he2, g1, w1a, w1b, row(W1_b), W2_w, row(W2_b), W3_w,
      row(W3_b), Win_w, row(Win_b), Wout_w, row(Wout_b), row(norm1_g),
      row(norm1_b), row(norm2_g), row(norm2_b), w11c)

    g2 = _sc_gather(p2, idx)

    wspecs2 = [
        _w_spec((H, H)), _w_spec((H, H)), _w_spec((1, H)),
        _w_spec((H, H)), _w_spec((1, H)), _w_spec((H, H)), _w_spec((1, H)),
        _w_spec((1, H)), _w_spec((1, H)),
    ]
    he_new = pl.pallas_call(
        _tc2_body,
        grid=(grid,),
        in_specs=[_node_spec(), _edge_spec(), _edge_spec()] + wspecs2,
        out_specs=_edge_spec(),
        out_shape=jax.ShapeDtypeStruct((M, H), jnp.float32),
    )(hv_new, he2, g2, w11a, w11b, row(W11_b), W12_w, row(W12_b),
      W13_w, row(W13_b), row(norm3_g), row(norm3_b))

    return hv_new.reshape(B, N, H), he_new.reshape(B, N, K, H)
